# baseline (speedup 1.0000x reference)
import functools

import jax
import jax.numpy as jnp
from jax import lax
from jax.experimental import pallas as pl
from jax.experimental.pallas import tpu as pltpu
from jax.experimental.pallas import tpu_sc as plsc

B = 2
TV = 1024
NWORK = 32
_SC_BUF = 131072


def _bn_apply(y, st_ref, g_ref, be_ref, n, rows, C):
    mean = st_ref[0] * (1.0 / n)
    var = st_ref[1] * (1.0 / n) - mean * mean
    scale = g_ref[0] * lax.rsqrt(var + 1e-5)
    shift = be_ref[0] - mean * scale
    y3 = y.reshape(rows, B, C)
    o = jnp.maximum(y3 * scale[None, None, :] + shift[None, None, :], 0.0)
    return o.reshape(rows, B * C)


def _conv_a_matmul(xprev, xenc, WT, WB, V, Cp, Ce, Co, bn=None, split=False):
    TV4 = TV // 4
    BCo = B * Co
    n_prev = float((V // 4) * B)

    def body(xp_ref, xe_ref, wt_ref, wb_ref, *refs):
        if bn is not None:
            st_ref, g_ref, be_ref = refs[:3]
            refs = refs[3:]
            xp = _bn_apply(xp_ref[...], st_ref, g_ref, be_ref, n_prev, TV4, Cp)
        else:
            xp = xp_ref[...]
        out_refs = refs
        us = []
        for b in range(B):
            xpb = xp[:, b * Cp:(b + 1) * Cp]
            xeb = xe_ref[b]
            p = jnp.dot(xpb, wt_ref[...], preferred_element_type=jnp.float32)
            p4 = jnp.broadcast_to(p[:, None, :],
                                  (TV4, 4, 3 * Co)).reshape(TV, 3 * Co)
            full = p4 + jnp.dot(xeb, wb_ref[...],
                                preferred_element_type=jnp.float32)
            us.append(full)
        parts = [us[b][:, k * Co:(k + 1) * Co] for k in range(3)
                 for b in range(B)]
        if split:
            for k in range(3):
                out_refs[k][...] = jnp.concatenate(parts[k * B:(k + 1) * B],
                                                   axis=1)
        else:
            out_refs[0][...] = jnp.concatenate(parts[:B], axis=1)
            out_refs[1][...] = jnp.concatenate(parts[B:], axis=1)

    widths = [BCo, BCo, BCo] if split else [BCo, 2 * BCo]
    in_specs = [
        pl.BlockSpec((TV4, B * Cp), lambda i: (i, 0)),
        pl.BlockSpec((B, TV, Ce), lambda i: (0, i, 0)),
        pl.BlockSpec((Cp, 3 * Co), lambda i: (0, 0)),
        pl.BlockSpec((Ce, 3 * Co), lambda i: (0, 0)),
    ]
    args = [xprev, xenc,
            jnp.concatenate([WT[0] - WT[2], WT[1], WT[2]], axis=1),
            jnp.concatenate([WB[0] - WB[2], WB[1], WB[2]], axis=1)]
    if bn is not None:
        stats, g, be = bn
        in_specs += [
            pl.BlockSpec((2, Cp), lambda i: (0, 0)),
            pl.BlockSpec((1, Cp), lambda i: (0, 0)),
            pl.BlockSpec((1, Cp), lambda i: (0, 0)),
        ]
        args += [stats, g.reshape(1, Cp), be.reshape(1, Cp)]
    return pl.pallas_call(
        body,
        grid=(V // TV,),
        in_specs=in_specs,
        out_specs=[pl.BlockSpec((TV, w), lambda i: (i, 0)) for w in widths],
        out_shape=[jax.ShapeDtypeStruct((V, w), jnp.float32) for w in widths],
    )(*args)


def _conv_b_matmul(y, stats, g, be, W, V, Cin, Co):
    BCo = B * Co
    n = float(V * B)

    def body(y_ref, st_ref, g_ref, be_ref, w_ref, u0_ref, u12_ref):
        zt = _bn_apply(y_ref[...], st_ref, g_ref, be_ref, n, TV, Cin)
        us = []
        for b in range(B):
            zb = zt[:, b * Cin:(b + 1) * Cin]
            us.append(jnp.dot(zb, w_ref[...],
                              preferred_element_type=jnp.float32))
        parts = [us[b][:, k * Co:(k + 1) * Co] for k in range(3)
                 for b in range(B)]
        u0_ref[...] = jnp.concatenate(parts[:B], axis=1)
        u12_ref[...] = jnp.concatenate(parts[B:], axis=1)

    return pl.pallas_call(
        body,
        grid=(V // TV,),
        in_specs=[
            pl.BlockSpec((TV, B * Cin), lambda i: (i, 0)),
            pl.BlockSpec((2, Cin), lambda i: (0, 0)),
            pl.BlockSpec((1, Cin), lambda i: (0, 0)),
            pl.BlockSpec((1, Cin), lambda i: (0, 0)),
            pl.BlockSpec((Cin, 3 * Co), lambda i: (0, 0)),
        ],
        out_specs=[
            pl.BlockSpec((TV, BCo), lambda i: (i, 0)),
            pl.BlockSpec((TV, 2 * BCo), lambda i: (i, 0)),
        ],
        out_shape=[
            jax.ShapeDtypeStruct((V, BCo), jnp.float32),
            jax.ShapeDtypeStruct((V, 2 * BCo), jnp.float32),
        ],
    )(y, stats, g.reshape(1, Cin), be.reshape(1, Cin),
      jnp.concatenate([W[0] - W[2], W[1], W[2]], axis=1))


def _combine_stats(u02, s, t, V, Co, dvals=None):
    BCo = B * Co
    Ws = s.shape[1]

    def body(*refs):
        if dvals is not None:
            u0_ref, s_ref, t_ref, d_ref, y_ref, st_ref = refs
            lu2 = s_ref[:, BCo:2 * BCo]
            t_full = t_ref[...] + d_ref[...] * lu2
        else:
            u0_ref, s_ref, t_ref, y_ref, st_ref = refs
            t_full = t_ref[...]
        y = u0_ref[...] + s_ref[:, :BCo] + 2.0 * t_full
        y_ref[...] = y
        y3 = y.reshape(TV, B, Co)
        ssum = jnp.sum(y3, axis=(0, 1))
        ssq = jnp.sum(y3 * y3, axis=(0, 1))
        part = jnp.stack([ssum, ssq], axis=0)

        @pl.when(pl.program_id(0) == 0)
        def _():
            st_ref[...] = jnp.zeros_like(st_ref)

        st_ref[...] += part

    in_specs = [
        pl.BlockSpec((TV, BCo), lambda i: (i, 0)),
        pl.BlockSpec((TV, Ws), lambda i: (i, 0)),
        pl.BlockSpec((TV, BCo), lambda i: (i, 0)),
    ]
    args = [u02, s, t]
    if dvals is not None:
        in_specs.append(pl.BlockSpec((TV, 1), lambda i: (i, 0)))
        args.append(dvals.reshape(V, 1))
    return pl.pallas_call(
        body,
        grid=(V // TV,),
        in_specs=in_specs,
        out_specs=[
            pl.BlockSpec((TV, BCo), lambda i: (i, 0)),
            pl.BlockSpec((2, Co), lambda i: (0, 0)),
        ],
        out_shape=[
            jax.ShapeDtypeStruct((V, BCo), jnp.float32),
            jax.ShapeDtypeStruct((2, Co), jnp.float32),
        ],
    )(*args)


def _final_proj(y, stats, g, be, wfull, bfin, V):
    n = float(V * B)

    def body(y_ref, st_ref, g_ref, be_ref, w_ref, b_ref, o_ref):
        x = _bn_apply(y_ref[...], st_ref, g_ref, be_ref, n, TV, 16)
        r = lax.dot_general(w_ref[...], x, (((1,), (1,)), ((), ())),
                            preferred_element_type=jnp.float32)
        o_ref[...] = r + b_ref[0, 0]

    return pl.pallas_call(
        body,
        grid=(V // TV,),
        in_specs=[
            pl.BlockSpec((TV, B * 16), lambda i: (i, 0)),
            pl.BlockSpec((2, 16), lambda i: (0, 0)),
            pl.BlockSpec((1, 16), lambda i: (0, 0)),
            pl.BlockSpec((1, 16), lambda i: (0, 0)),
            pl.BlockSpec((B, B * 16), lambda i: (0, 0)),
            pl.BlockSpec((1, 1), lambda i: (0, 0)),
        ],
        out_specs=pl.BlockSpec((B, TV), lambda i: (0, i)),
        out_shape=jax.ShapeDtypeStruct((B, V), jnp.float32),
    )(y, stats, g.reshape(1, 16), be.reshape(1, 16), wfull, bfin)


def _spmm_plan(V, W):
    rpw = V // NWORK
    for R in (128, 64, 32, 16, 8):
        if R * 8 * W * 4 <= _SC_BUF and rpw % R == 0:
            break
    G = R * 8
    colw = min(G, 128)
    return rpw, R, G, colw, G // colw


def _spmm_sc(z, cols, vals, dvals, V, W, col_off=0, diag=True):
    rpw, R, G, colw, nG = _spmm_plan(V, W)
    nchunk = rpw // R
    assert nchunk % 2 == 0 and G % colw == 0
    wide = col_off > 0 or z.shape[1] != W
    csl = (pl.ds(col_off, W),) if wide else ()
    mesh = plsc.VectorSubcoreMesh(core_axis_name="c", subcore_axis_name="s")

    @functools.partial(
        pl.kernel, mesh=mesh,
        compiler_params=pltpu.CompilerParams(needs_layout_passes=False,
                                             use_tc_tiling_on_sc=False),
        out_type=jax.ShapeDtypeStruct((V, W), jnp.float32),
        scratch_types=(
            [pltpu.VMEM((rpw * 8,), jnp.int32),
             pltpu.VMEM((rpw * 8,), jnp.float32),
             pltpu.VMEM((rpw,), jnp.float32)]
            + [pltpu.VMEM((G, W), jnp.float32) for _ in range(2)]
            + [pltpu.VMEM((R, W), jnp.float32) for _ in range(4)]
            + [pltpu.SemaphoreType.DMA for _ in range(4)]
        ),
    )
    def k(z_hbm, cols_hbm, vals_hbm, dv_hbm, out_hbm,
          colv, vv, dvv, gat0, gat1, zd0, zd1, ov0, ov1,
          sem0, sem1, osem0, osem1):
        wid = lax.axis_index("s") * 2 + lax.axis_index("c")
        wbase = pl.multiple_of(wid * rpw, 8)
        gat = (gat0, gat1)
        zd = (zd0, zd1)
        ov = (ov0, ov1)
        sem = (sem0, sem1)
        osem = (osem0, osem1)

        pltpu.sync_copy(cols_hbm.at[pl.ds(pl.multiple_of(wbase * 8, 64),
                                          rpw * 8)], colv)
        pltpu.sync_copy(vals_hbm.at[pl.ds(pl.multiple_of(wbase * 8, 64),
                                          rpw * 8)], vv)
        if diag:
            pltpu.sync_copy(dv_hbm.at[pl.ds(wbase, rpw)], dvv)

        def fire(ci, b):
            base = pl.multiple_of(wbase + ci * R, 8)
            for g in range(nG):
                pltpu.async_copy(
                    z_hbm.at[(colv.at[pl.ds(ci * G + g * colw, colw)],) + csl],
                    gat[b].at[pl.ds(g * colw, colw)], sem[b])
            if diag:
                pltpu.async_copy(z_hbm.at[(pl.ds(base, R),) + csl],
                                 zd[b], sem[b])

        def drain(ci, b):
            base = pl.multiple_of(wbase + ci * R, 8)
            for g in range(nG):
                pltpu.make_async_copy(
                    z_hbm.at[(colv.at[pl.ds(ci * G + g * colw, colw)],) + csl],
                    gat[b].at[pl.ds(g * colw, colw)], sem[b]).wait()
            if diag:
                pltpu.make_async_copy(z_hbm.at[(pl.ds(base, R),) + csl],
                                      zd[b], sem[b]).wait()

        fire(0, 0)

        def pair(ii, _):
            for b in range(2):
                ci = ii * 2 + b
                nb = 1 - b
                base = pl.multiple_of(wbase + ci * R, 8)

                @pl.when(ci + 1 < nchunk)
                def _():
                    fire(ci + 1, nb)

                drain(ci, b)

                @pl.when(ci >= 2)
                def _():
                    pltpu.make_async_copy(ov[b], out_hbm.at[pl.ds(base, R)],
                                          osem[b]).wait()

                gb, zb, ob = gat[b], zd[b], ov[b]

                def row(r, _):
                    e = (ci * R + r) * 8
                    vjs = [plsc.load_gather(
                        vv, [jnp.full((16,), e + j, jnp.int32)])
                        for j in range(8)]
                    if diag:
                        dv = plsc.load_gather(
                            dvv, [jnp.full((16,), ci * R + r, jnp.int32)])
                    for wt in range(W // 16):
                        sl = pl.ds(wt * 16, 16)
                        if diag:
                            acc = dv * zb[r, sl]
                        else:
                            acc = vjs[0] * gb[r * 8, sl]
                        for j in range(0 if diag else 1, 8):
                            acc = acc + vjs[j] * gb[r * 8 + j, sl]
                        ob[r, sl] = acc
                    return 0

                lax.fori_loop(0, R, row, 0)
                pltpu.async_copy(ov[b], out_hbm.at[pl.ds(base, R)], osem[b])
            return 0

        lax.fori_loop(0, nchunk // 2, pair, 0)
        for b in range(2):
            last = pl.multiple_of(wbase + (nchunk - 2 + b) * R, 8)
            pltpu.make_async_copy(ov[b], out_hbm.at[pl.ds(last, R)],
                                  osem[b]).wait()

    return k(z, cols, vals, dvals)


def kernel(x_enc0, x_enc1, x_enc2, x_enc3, lap1_idx, lap1_val, lap2_idx,
           lap2_val, lap3_idx, lap3_val, W1a, b1a, g1a, be1a, W1b, b1b, g1b,
           be1b, W2a, b2a, g2a, be2a, W2b, b2b, g2b, be2b, W3a, b3a, g3a,
           be3a, W3b, b3b, g3b, be3b, Wfin, bfin):
    x = x_enc0.transpose(1, 0, 2).reshape(768, B * 256)
    bn_prev = None
    specs = [
        (3072, x_enc1, lap1_idx, lap1_val, W1a, g1a, be1a, W1b, g1b, be1b),
        (12288, x_enc2, lap2_idx, lap2_val, W2a, g2a, be2a, W2b, g2b, be2b),
        (49152, x_enc3, lap3_idx, lap3_val, W3a, g3a, be3a, W3b, g3b, be3b),
    ]
    for V, xe, lidx, lval, Wa, ga, bea, Wb, gb, beb in specs:
        Cp = x.shape[1] // B
        Ce = xe.shape[2]
        Coa, Cob = Wa.shape[2], Wb.shape[2]
        cols = lidx[1, :8 * V]
        vals = lval[:8 * V]
        dvals = lval[8 * V:]

        split = 2 * B * Coa > 512
        if split:
            u02, u1, u2 = _conv_a_matmul(x, xe, Wa[:, :Cp, :], Wa[:, Cp:, :],
                                         V, Cp, Ce, Coa, bn=bn_prev,
                                         split=True)
            s = _spmm_sc(u1, cols, vals, dvals, V, B * Coa)
            s2 = _spmm_sc(u2, cols, vals, dvals, V, B * Coa)
            t = _spmm_sc(s2, cols, vals, dvals, V, B * Coa)
            y, stats = _combine_stats(u02, s, t, V, Coa)
        else:
            u02, u12 = _conv_a_matmul(x, xe, Wa[:, :Cp, :], Wa[:, Cp:, :],
                                      V, Cp, Ce, Coa, bn=bn_prev)
            s = _spmm_sc(u12, cols, vals, dvals, V, 2 * B * Coa)
            t = _spmm_sc(s[:, B * Coa:], cols, vals, dvals, V, B * Coa,
                         diag=False)
            y, stats = _combine_stats(u02, s, t, V, Coa, dvals=dvals)

        u02, u12 = _conv_b_matmul(y, stats, ga, bea, Wb, V, Coa, Cob)
        s = _spmm_sc(u12, cols, vals, dvals, V, 2 * B * Cob)
        t = _spmm_sc(s[:, B * Cob:], cols, vals, dvals, V, B * Cob,
                     diag=False)
        y, stats = _combine_stats(u02, s, t, V, Cob, dvals=dvals)
        x = y
        bn_prev = (stats, gb, beb)

    V3 = 49152
    wfull = jnp.zeros((B, B * 16), jnp.float32)
    for b in range(B):
        wfull = wfull.at[b, b * 16:(b + 1) * 16].set(Wfin[0])
    stats3, g3, be3 = bn_prev
    out = _final_proj(x, stats3, g3, be3, wfull, bfin.reshape(1, 1), V3)
    return out.reshape(B, 1, V3)


# speedup vs baseline: 1.0117x; 1.0117x over previous
import functools

import jax
import jax.numpy as jnp
from jax import lax
from jax.experimental import pallas as pl
from jax.experimental.pallas import tpu as pltpu
from jax.experimental.pallas import tpu_sc as plsc

B = 2
TV = 1024
NWORK = 32
_SC_BUF = 131072


def _bn_apply(y, st_ref, g_ref, be_ref, n, rows, C):
    mean = st_ref[0] * (1.0 / n)
    var = st_ref[1] * (1.0 / n) - mean * mean
    scale = g_ref[0] * lax.rsqrt(var + 1e-5)
    shift = be_ref[0] - mean * scale
    y3 = y.reshape(rows, B, C)
    o = jnp.maximum(y3 * scale[None, None, :] + shift[None, None, :], 0.0)
    return o.reshape(rows, B * C)


def _conv_a_matmul(xprev, xenc, WT, WB, V, Cp, Ce, Co, bn=None, split=False):
    TV4 = TV // 4
    BCo = B * Co
    n_prev = float((V // 4) * B)

    def body(xp_ref, xe_ref, wt_ref, wb_ref, *refs):
        if bn is not None:
            st_ref, g_ref, be_ref = refs[:3]
            refs = refs[3:]
            xp = _bn_apply(xp_ref[...], st_ref, g_ref, be_ref, n_prev, TV4, Cp)
        else:
            xp = xp_ref[...]
        out_refs = refs
        parts = []
        for k in range(3):
            for b in range(B):
                xpb = xp[:, b * Cp:(b + 1) * Cp]
                xeb = xe_ref[b]
                p = jnp.dot(xpb, wt_ref[k], preferred_element_type=jnp.float32)
                p4 = jnp.broadcast_to(p[:, None, :],
                                      (TV4, 4, Co)).reshape(TV, Co)
                parts.append(p4 + jnp.dot(xeb, wb_ref[k],
                                          preferred_element_type=jnp.float32))
        if split:
            for k in range(3):
                out_refs[k][...] = jnp.concatenate(parts[k * B:(k + 1) * B],
                                                   axis=1)
        else:
            out_refs[0][...] = jnp.concatenate(parts[:B], axis=1)
            out_refs[1][...] = jnp.concatenate(parts[B:], axis=1)

    widths = [BCo, BCo, BCo] if split else [BCo, 2 * BCo]
    in_specs = [
        pl.BlockSpec((TV4, B * Cp), lambda i: (i, 0)),
        pl.BlockSpec((B, TV, Ce), lambda i: (0, i, 0)),
        pl.BlockSpec((3, Cp, Co), lambda i: (0, 0, 0)),
        pl.BlockSpec((3, Ce, Co), lambda i: (0, 0, 0)),
    ]
    args = [xprev, xenc,
            jnp.stack([WT[0] - WT[2], WT[1], WT[2]]),
            jnp.stack([WB[0] - WB[2], WB[1], WB[2]])]
    if bn is not None:
        stats, g, be = bn
        in_specs += [
            pl.BlockSpec((2, Cp), lambda i: (0, 0)),
            pl.BlockSpec((1, Cp), lambda i: (0, 0)),
            pl.BlockSpec((1, Cp), lambda i: (0, 0)),
        ]
        args += [stats, g.reshape(1, Cp), be.reshape(1, Cp)]
    return pl.pallas_call(
        body,
        grid=(V // TV,),
        in_specs=in_specs,
        out_specs=[pl.BlockSpec((TV, w), lambda i: (i, 0)) for w in widths],
        out_shape=[jax.ShapeDtypeStruct((V, w), jnp.float32) for w in widths],
    )(*args)


def _conv_b_matmul(y, stats, g, be, W, V, Cin, Co):
    BCo = B * Co
    n = float(V * B)

    def body(y_ref, st_ref, g_ref, be_ref, w_ref, u0_ref, u12_ref):
        zt = _bn_apply(y_ref[...], st_ref, g_ref, be_ref, n, TV, Cin)
        parts = []
        for k in range(3):
            for b in range(B):
                zb = zt[:, b * Cin:(b + 1) * Cin]
                parts.append(jnp.dot(zb, w_ref[k],
                                     preferred_element_type=jnp.float32))
        u0_ref[...] = jnp.concatenate(parts[:B], axis=1)
        u12_ref[...] = jnp.concatenate(parts[B:], axis=1)

    return pl.pallas_call(
        body,
        grid=(V // TV,),
        in_specs=[
            pl.BlockSpec((TV, B * Cin), lambda i: (i, 0)),
            pl.BlockSpec((2, Cin), lambda i: (0, 0)),
            pl.BlockSpec((1, Cin), lambda i: (0, 0)),
            pl.BlockSpec((1, Cin), lambda i: (0, 0)),
            pl.BlockSpec((3, Cin, Co), lambda i: (0, 0, 0)),
        ],
        out_specs=[
            pl.BlockSpec((TV, BCo), lambda i: (i, 0)),
            pl.BlockSpec((TV, 2 * BCo), lambda i: (i, 0)),
        ],
        out_shape=[
            jax.ShapeDtypeStruct((V, BCo), jnp.float32),
            jax.ShapeDtypeStruct((V, 2 * BCo), jnp.float32),
        ],
    )(y, stats, g.reshape(1, Cin), be.reshape(1, Cin),
      jnp.stack([W[0] - W[2], W[1], W[2]]))


def _combine_stats(u02, s, t, V, Co, dvals=None):
    BCo = B * Co
    Ws = s.shape[1]

    def body(*refs):
        if dvals is not None:
            u0_ref, s_ref, t_ref, d_ref, y_ref, st_ref = refs
            lu2 = s_ref[:, BCo:2 * BCo]
            t_full = t_ref[...] + d_ref[...] * lu2
        else:
            u0_ref, s_ref, t_ref, y_ref, st_ref = refs
            t_full = t_ref[...]
        y = u0_ref[...] + s_ref[:, :BCo] + 2.0 * t_full
        y_ref[...] = y
        y3 = y.reshape(TV, B, Co)
        ssum = jnp.sum(y3, axis=(0, 1))
        ssq = jnp.sum(y3 * y3, axis=(0, 1))
        part = jnp.stack([ssum, ssq], axis=0)

        @pl.when(pl.program_id(0) == 0)
        def _():
            st_ref[...] = jnp.zeros_like(st_ref)

        st_ref[...] += part

    in_specs = [
        pl.BlockSpec((TV, BCo), lambda i: (i, 0)),
        pl.BlockSpec((TV, Ws), lambda i: (i, 0)),
        pl.BlockSpec((TV, BCo), lambda i: (i, 0)),
    ]
    args = [u02, s, t]
    if dvals is not None:
        in_specs.append(pl.BlockSpec((TV, 1), lambda i: (i, 0)))
        args.append(dvals.reshape(V, 1))
    return pl.pallas_call(
        body,
        grid=(V // TV,),
        in_specs=in_specs,
        out_specs=[
            pl.BlockSpec((TV, BCo), lambda i: (i, 0)),
            pl.BlockSpec((2, Co), lambda i: (0, 0)),
        ],
        out_shape=[
            jax.ShapeDtypeStruct((V, BCo), jnp.float32),
            jax.ShapeDtypeStruct((2, Co), jnp.float32),
        ],
    )(*args)


def _final_proj(y, stats, g, be, wfull, bfin, V):
    n = float(V * B)

    def body(y_ref, st_ref, g_ref, be_ref, w_ref, b_ref, o_ref):
        x = _bn_apply(y_ref[...], st_ref, g_ref, be_ref, n, TV, 16)
        r = lax.dot_general(w_ref[...], x, (((1,), (1,)), ((), ())),
                            preferred_element_type=jnp.float32)
        o_ref[...] = r + b_ref[0, 0]

    return pl.pallas_call(
        body,
        grid=(V // TV,),
        in_specs=[
            pl.BlockSpec((TV, B * 16), lambda i: (i, 0)),
            pl.BlockSpec((2, 16), lambda i: (0, 0)),
            pl.BlockSpec((1, 16), lambda i: (0, 0)),
            pl.BlockSpec((1, 16), lambda i: (0, 0)),
            pl.BlockSpec((B, B * 16), lambda i: (0, 0)),
            pl.BlockSpec((1, 1), lambda i: (0, 0)),
        ],
        out_specs=pl.BlockSpec((B, TV), lambda i: (0, i)),
        out_shape=jax.ShapeDtypeStruct((B, V), jnp.float32),
    )(y, stats, g.reshape(1, 16), be.reshape(1, 16), wfull, bfin)


def _spmm_plan(V, W):
    rpw = V // NWORK
    for R in (128, 64, 32, 16, 8):
        if R * 8 * W * 4 <= _SC_BUF and rpw % R == 0:
            break
    G = R * 8
    colw = min(G, 128)
    return rpw, R, G, colw, G // colw


def _spmm_sc(z, cols, vals, dvals, V, W, col_off=0, diag=True):
    rpw, R, G, colw, nG = _spmm_plan(V, W)
    nchunk = rpw // R
    assert nchunk % 2 == 0 and G % colw == 0
    wide = col_off > 0 or z.shape[1] != W
    csl = (pl.ds(col_off, W),) if wide else ()
    mesh = plsc.VectorSubcoreMesh(core_axis_name="c", subcore_axis_name="s")

    @functools.partial(
        pl.kernel, mesh=mesh,
        compiler_params=pltpu.CompilerParams(needs_layout_passes=False,
                                             use_tc_tiling_on_sc=False),
        out_type=jax.ShapeDtypeStruct((V, W), jnp.float32),
        scratch_types=(
            [pltpu.VMEM((rpw * 8,), jnp.int32),
             pltpu.VMEM((rpw * 8,), jnp.float32),
             pltpu.VMEM((rpw,), jnp.float32)]
            + [pltpu.VMEM((G, W), jnp.float32) for _ in range(2)]
            + [pltpu.VMEM((R, W), jnp.float32) for _ in range(4)]
            + [pltpu.SemaphoreType.DMA for _ in range(4)]
        ),
    )
    def k(z_hbm, cols_hbm, vals_hbm, dv_hbm, out_hbm,
          colv, vv, dvv, gat0, gat1, zd0, zd1, ov0, ov1,
          sem0, sem1, osem0, osem1):
        wid = lax.axis_index("s") * 2 + lax.axis_index("c")
        wbase = pl.multiple_of(wid * rpw, 8)
        gat = (gat0, gat1)
        zd = (zd0, zd1)
        ov = (ov0, ov1)
        sem = (sem0, sem1)
        osem = (osem0, osem1)

        pltpu.sync_copy(cols_hbm.at[pl.ds(pl.multiple_of(wbase * 8, 64),
                                          rpw * 8)], colv)
        pltpu.sync_copy(vals_hbm.at[pl.ds(pl.multiple_of(wbase * 8, 64),
                                          rpw * 8)], vv)
        if diag:
            pltpu.sync_copy(dv_hbm.at[pl.ds(wbase, rpw)], dvv)

        def fire(ci, b):
            base = pl.multiple_of(wbase + ci * R, 8)
            for g in range(nG):
                pltpu.async_copy(
                    z_hbm.at[(colv.at[pl.ds(ci * G + g * colw, colw)],) + csl],
                    gat[b].at[pl.ds(g * colw, colw)], sem[b])
            if diag:
                pltpu.async_copy(z_hbm.at[(pl.ds(base, R),) + csl],
                                 zd[b], sem[b])

        def drain(ci, b):
            base = pl.multiple_of(wbase + ci * R, 8)
            for g in range(nG):
                pltpu.make_async_copy(
                    z_hbm.at[(colv.at[pl.ds(ci * G + g * colw, colw)],) + csl],
                    gat[b].at[pl.ds(g * colw, colw)], sem[b]).wait()
            if diag:
                pltpu.make_async_copy(z_hbm.at[(pl.ds(base, R),) + csl],
                                      zd[b], sem[b]).wait()

        fire(0, 0)

        def pair(ii, _):
            for b in range(2):
                ci = ii * 2 + b
                nb = 1 - b
                base = pl.multiple_of(wbase + ci * R, 8)

                @pl.when(ci + 1 < nchunk)
                def _():
                    fire(ci + 1, nb)

                drain(ci, b)

                @pl.when(ci >= 2)
                def _():
                    pltpu.make_async_copy(ov[b], out_hbm.at[pl.ds(base, R)],
                                          osem[b]).wait()

                gb, zb, ob = gat[b], zd[b], ov[b]

                def row(r, _):
                    e = (ci * R + r) * 8
                    vjs = [plsc.load_gather(
                        vv, [jnp.full((16,), e + j, jnp.int32)])
                        for j in range(8)]
                    if diag:
                        dv = plsc.load_gather(
                            dvv, [jnp.full((16,), ci * R + r, jnp.int32)])
                    for wt in range(W // 16):
                        sl = pl.ds(wt * 16, 16)
                        if diag:
                            acc = dv * zb[r, sl]
                        else:
                            acc = vjs[0] * gb[r * 8, sl]
                        for j in range(0 if diag else 1, 8):
                            acc = acc + vjs[j] * gb[r * 8 + j, sl]
                        ob[r, sl] = acc
                    return 0

                lax.fori_loop(0, R, row, 0)
                pltpu.async_copy(ov[b], out_hbm.at[pl.ds(base, R)], osem[b])
            return 0

        lax.fori_loop(0, nchunk // 2, pair, 0)
        for b in range(2):
            last = pl.multiple_of(wbase + (nchunk - 2 + b) * R, 8)
            pltpu.make_async_copy(ov[b], out_hbm.at[pl.ds(last, R)],
                                  osem[b]).wait()

    return k(z, cols, vals, dvals)


def kernel(x_enc0, x_enc1, x_enc2, x_enc3, lap1_idx, lap1_val, lap2_idx,
           lap2_val, lap3_idx, lap3_val, W1a, b1a, g1a, be1a, W1b, b1b, g1b,
           be1b, W2a, b2a, g2a, be2a, W2b, b2b, g2b, be2b, W3a, b3a, g3a,
           be3a, W3b, b3b, g3b, be3b, Wfin, bfin):
    x = x_enc0.transpose(1, 0, 2).reshape(768, B * 256)
    bn_prev = None
    specs = [
        (3072, x_enc1, lap1_idx, lap1_val, W1a, g1a, be1a, W1b, g1b, be1b),
        (12288, x_enc2, lap2_idx, lap2_val, W2a, g2a, be2a, W2b, g2b, be2b),
        (49152, x_enc3, lap3_idx, lap3_val, W3a, g3a, be3a, W3b, g3b, be3b),
    ]
    for V, xe, lidx, lval, Wa, ga, bea, Wb, gb, beb in specs:
        Cp = x.shape[1] // B
        Ce = xe.shape[2]
        Coa, Cob = Wa.shape[2], Wb.shape[2]
        cols = lidx[1, :8 * V]
        vals = lval[:8 * V]
        dvals = lval[8 * V:]

        split = 2 * B * Coa > 512
        if split:
            u02, u1, u2 = _conv_a_matmul(x, xe, Wa[:, :Cp, :], Wa[:, Cp:, :],
                                         V, Cp, Ce, Coa, bn=bn_prev,
                                         split=True)
            s = _spmm_sc(u1, cols, vals, dvals, V, B * Coa)
            s2 = _spmm_sc(u2, cols, vals, dvals, V, B * Coa)
            t = _spmm_sc(s2, cols, vals, dvals, V, B * Coa)
            y, stats = _combine_stats(u02, s, t, V, Coa)
        else:
            u02, u12 = _conv_a_matmul(x, xe, Wa[:, :Cp, :], Wa[:, Cp:, :],
                                      V, Cp, Ce, Coa, bn=bn_prev)
            s = _spmm_sc(u12, cols, vals, dvals, V, 2 * B * Coa)
            t = _spmm_sc(s[:, B * Coa:], cols, vals, dvals, V, B * Coa)
            y, stats = _combine_stats(u02, s, t, V, Coa)

        u02, u12 = _conv_b_matmul(y, stats, ga, bea, Wb, V, Coa, Cob)
        s = _spmm_sc(u12, cols, vals, dvals, V, 2 * B * Cob)
        t = _spmm_sc(s[:, B * Cob:], cols, vals, dvals, V, B * Cob)
        y, stats = _combine_stats(u02, s, t, V, Cob)
        x = y
        bn_prev = (stats, gb, beb)

    V3 = 49152
    wfull = jnp.zeros((B, B * 16), jnp.float32)
    for b in range(B):
        wfull = wfull.at[b, b * 16:(b + 1) * 16].set(Wfin[0])
    stats3, g3, be3 = bn_prev
    out = _final_proj(x, stats3, g3, be3, wfull, bfin.reshape(1, 1), V3)
    return out.reshape(B, 1, V3)


# speedup vs baseline: 1.0904x; 1.0778x over previous
import functools

import jax
import jax.numpy as jnp
from jax import lax
from jax.experimental import pallas as pl
from jax.experimental.pallas import tpu as pltpu
from jax.experimental.pallas import tpu_sc as plsc

B = 2
TV = 1024
NWORK = 32
_SC_BUF = 131072


def _bn_apply(y, st_ref, g_ref, be_ref, n, rows, C):
    mean = st_ref[0] * (1.0 / n)
    var = st_ref[1] * (1.0 / n) - mean * mean
    scale = g_ref[0] * lax.rsqrt(var + 1e-5)
    shift = be_ref[0] - mean * scale
    y3 = y.reshape(rows, B, C)
    o = jnp.maximum(y3 * scale[None, None, :] + shift[None, None, :], 0.0)
    return o.reshape(rows, B * C)


def _conv_a_matmul(xprev, xenc, WT, WB, V, Cp, Ce, Co, bn=None, split=False):
    TV4 = TV // 4
    BCo = B * Co
    n_prev = float((V // 4) * B)

    def body(xp_ref, xe_ref, wt_ref, wb_ref, *refs):
        if bn is not None:
            st_ref, g_ref, be_ref = refs[:3]
            refs = refs[3:]
            xp = _bn_apply(xp_ref[...], st_ref, g_ref, be_ref, n_prev, TV4, Cp)
        else:
            xp = xp_ref[...]
        out_refs = refs
        parts = []
        for k in range(3):
            for b in range(B):
                xpb = xp[:, b * Cp:(b + 1) * Cp]
                xeb = xe_ref[b]
                p = jnp.dot(xpb, wt_ref[k], preferred_element_type=jnp.float32)
                p4 = jnp.broadcast_to(p[:, None, :],
                                      (TV4, 4, Co)).reshape(TV, Co)
                parts.append(p4 + jnp.dot(xeb, wb_ref[k],
                                          preferred_element_type=jnp.float32))
        if split:
            for k in range(3):
                out_refs[k][...] = jnp.concatenate(parts[k * B:(k + 1) * B],
                                                   axis=1)
        else:
            out_refs[0][...] = jnp.concatenate(parts[:B], axis=1)
            out_refs[1][...] = jnp.concatenate(parts[B:], axis=1)

    widths = [BCo, BCo, BCo] if split else [BCo, 2 * BCo]
    in_specs = [
        pl.BlockSpec((TV4, B * Cp), lambda i: (i, 0)),
        pl.BlockSpec((B, TV, Ce), lambda i: (0, i, 0)),
        pl.BlockSpec((3, Cp, Co), lambda i: (0, 0, 0)),
        pl.BlockSpec((3, Ce, Co), lambda i: (0, 0, 0)),
    ]
    args = [xprev, xenc,
            jnp.stack([WT[0] - WT[2], WT[1], WT[2]]),
            jnp.stack([WB[0] - WB[2], WB[1], WB[2]])]
    if bn is not None:
        stats, g, be = bn
        in_specs += [
            pl.BlockSpec((2, Cp), lambda i: (0, 0)),
            pl.BlockSpec((1, Cp), lambda i: (0, 0)),
            pl.BlockSpec((1, Cp), lambda i: (0, 0)),
        ]
        args += [stats, g.reshape(1, Cp), be.reshape(1, Cp)]
    return pl.pallas_call(
        body,
        grid=(V // TV,),
        in_specs=in_specs,
        out_specs=[pl.BlockSpec((TV, w), lambda i: (i, 0)) for w in widths],
        out_shape=[jax.ShapeDtypeStruct((V, w), jnp.float32) for w in widths],
    )(*args)


def _conv_b_matmul(y, stats, g, be, W, V, Cin, Co):
    BCo = B * Co
    n = float(V * B)

    def body(y_ref, st_ref, g_ref, be_ref, w_ref, u0_ref, u12_ref):
        zt = _bn_apply(y_ref[...], st_ref, g_ref, be_ref, n, TV, Cin)
        parts = []
        for k in range(3):
            for b in range(B):
                zb = zt[:, b * Cin:(b + 1) * Cin]
                parts.append(jnp.dot(zb, w_ref[k],
                                     preferred_element_type=jnp.float32))
        u0_ref[...] = jnp.concatenate(parts[:B], axis=1)
        u12_ref[...] = jnp.concatenate(parts[B:], axis=1)

    return pl.pallas_call(
        body,
        grid=(V // TV,),
        in_specs=[
            pl.BlockSpec((TV, B * Cin), lambda i: (i, 0)),
            pl.BlockSpec((2, Cin), lambda i: (0, 0)),
            pl.BlockSpec((1, Cin), lambda i: (0, 0)),
            pl.BlockSpec((1, Cin), lambda i: (0, 0)),
            pl.BlockSpec((3, Cin, Co), lambda i: (0, 0, 0)),
        ],
        out_specs=[
            pl.BlockSpec((TV, BCo), lambda i: (i, 0)),
            pl.BlockSpec((TV, 2 * BCo), lambda i: (i, 0)),
        ],
        out_shape=[
            jax.ShapeDtypeStruct((V, BCo), jnp.float32),
            jax.ShapeDtypeStruct((V, 2 * BCo), jnp.float32),
        ],
    )(y, stats, g.reshape(1, Cin), be.reshape(1, Cin),
      jnp.stack([W[0] - W[2], W[1], W[2]]))


def _combine_stats(u02, s, t, V, Co, dvals=None):
    BCo = B * Co
    Ws = s.shape[1]

    def body(*refs):
        if dvals is not None:
            u0_ref, s_ref, t_ref, d_ref, y_ref, st_ref = refs
            lu2 = s_ref[:, BCo:2 * BCo]
            t_full = t_ref[...] + d_ref[...] * lu2
        else:
            u0_ref, s_ref, t_ref, y_ref, st_ref = refs
            t_full = t_ref[...]
        y = u0_ref[...] + s_ref[:, :BCo] + 2.0 * t_full
        y_ref[...] = y
        y3 = y.reshape(TV, B, Co)
        ssum = jnp.sum(y3, axis=(0, 1))
        ssq = jnp.sum(y3 * y3, axis=(0, 1))
        part = jnp.stack([ssum, ssq], axis=0)

        @pl.when(pl.program_id(0) == 0)
        def _():
            st_ref[...] = jnp.zeros_like(st_ref)

        st_ref[...] += part

    in_specs = [
        pl.BlockSpec((TV, BCo), lambda i: (i, 0)),
        pl.BlockSpec((TV, Ws), lambda i: (i, 0)),
        pl.BlockSpec((TV, BCo), lambda i: (i, 0)),
    ]
    args = [u02, s, t]
    if dvals is not None:
        in_specs.append(pl.BlockSpec((TV, 1), lambda i: (i, 0)))
        args.append(dvals.reshape(V, 1))
    return pl.pallas_call(
        body,
        grid=(V // TV,),
        in_specs=in_specs,
        out_specs=[
            pl.BlockSpec((TV, BCo), lambda i: (i, 0)),
            pl.BlockSpec((2, Co), lambda i: (0, 0)),
        ],
        out_shape=[
            jax.ShapeDtypeStruct((V, BCo), jnp.float32),
            jax.ShapeDtypeStruct((2, Co), jnp.float32),
        ],
    )(*args)


def _final_proj(y, stats, g, be, wfull, bfin, V):
    n = float(V * B)

    def body(y_ref, st_ref, g_ref, be_ref, w_ref, b_ref, o_ref):
        x = _bn_apply(y_ref[...], st_ref, g_ref, be_ref, n, TV, 16)
        r = lax.dot_general(w_ref[...], x, (((1,), (1,)), ((), ())),
                            preferred_element_type=jnp.float32)
        o_ref[...] = r + b_ref[0, 0]

    return pl.pallas_call(
        body,
        grid=(V // TV,),
        in_specs=[
            pl.BlockSpec((TV, B * 16), lambda i: (i, 0)),
            pl.BlockSpec((2, 16), lambda i: (0, 0)),
            pl.BlockSpec((1, 16), lambda i: (0, 0)),
            pl.BlockSpec((1, 16), lambda i: (0, 0)),
            pl.BlockSpec((B, B * 16), lambda i: (0, 0)),
            pl.BlockSpec((1, 1), lambda i: (0, 0)),
        ],
        out_specs=pl.BlockSpec((B, TV), lambda i: (0, i)),
        out_shape=jax.ShapeDtypeStruct((B, V), jnp.float32),
    )(y, stats, g.reshape(1, 16), be.reshape(1, 16), wfull, bfin)


def _spmm_plan(V, W):
    rpw = V // NWORK
    for R in (128, 64, 32, 16, 8):
        if R * 8 * W * 4 <= _SC_BUF and rpw % R == 0:
            break
    G = R * 8
    colw = min(G, 128)
    return rpw, R, G, colw, G // colw


def _spmm_sc(z, cols, vals, dvals, V, W, col_off=0, diag=True):
    rpw, R, G, colw, nG = _spmm_plan(V, W)
    nchunk = rpw // R
    assert nchunk % 2 == 0 and G % colw == 0
    wide = col_off > 0 or z.shape[1] != W
    csl = (pl.ds(col_off, W),) if wide else ()
    mesh = plsc.VectorSubcoreMesh(core_axis_name="c", subcore_axis_name="s")

    @functools.partial(
        pl.kernel, mesh=mesh,
        compiler_params=pltpu.CompilerParams(
            needs_layout_passes=False,
            use_tc_tiling_on_sc=(W % 128 == 0)),
        out_type=jax.ShapeDtypeStruct((V, W), jnp.float32),
        scratch_types=(
            [pltpu.VMEM((rpw * 8,), jnp.int32),
             pltpu.VMEM((rpw * 8,), jnp.float32),
             pltpu.VMEM((rpw,), jnp.float32)]
            + [pltpu.VMEM((G, W), jnp.float32) for _ in range(2)]
            + [pltpu.VMEM((R, W), jnp.float32) for _ in range(4)]
            + [pltpu.SemaphoreType.DMA for _ in range(4)]
        ),
    )
    def k(z_hbm, cols_hbm, vals_hbm, dv_hbm, out_hbm,
          colv, vv, dvv, gat0, gat1, zd0, zd1, ov0, ov1,
          sem0, sem1, osem0, osem1):
        wid = lax.axis_index("s") * 2 + lax.axis_index("c")
        wbase = pl.multiple_of(wid * rpw, 8)
        gat = (gat0, gat1)
        zd = (zd0, zd1)
        ov = (ov0, ov1)
        sem = (sem0, sem1)
        osem = (osem0, osem1)

        pltpu.sync_copy(cols_hbm.at[pl.ds(pl.multiple_of(wbase * 8, 64),
                                          rpw * 8)], colv)
        pltpu.sync_copy(vals_hbm.at[pl.ds(pl.multiple_of(wbase * 8, 64),
                                          rpw * 8)], vv)
        if diag:
            pltpu.sync_copy(dv_hbm.at[pl.ds(wbase, rpw)], dvv)

        def fire(ci, b):
            base = pl.multiple_of(wbase + ci * R, 8)
            for g in range(nG):
                pltpu.async_copy(
                    z_hbm.at[(colv.at[pl.ds(ci * G + g * colw, colw)],) + csl],
                    gat[b].at[pl.ds(g * colw, colw)], sem[b])
            if diag:
                pltpu.async_copy(z_hbm.at[(pl.ds(base, R),) + csl],
                                 zd[b], sem[b])

        def drain(ci, b):
            base = pl.multiple_of(wbase + ci * R, 8)
            for g in range(nG):
                pltpu.make_async_copy(
                    z_hbm.at[(colv.at[pl.ds(ci * G + g * colw, colw)],) + csl],
                    gat[b].at[pl.ds(g * colw, colw)], sem[b]).wait()
            if diag:
                pltpu.make_async_copy(z_hbm.at[(pl.ds(base, R),) + csl],
                                      zd[b], sem[b]).wait()

        fire(0, 0)

        def pair(ii, _):
            for b in range(2):
                ci = ii * 2 + b
                nb = 1 - b
                base = pl.multiple_of(wbase + ci * R, 8)

                @pl.when(ci + 1 < nchunk)
                def _():
                    fire(ci + 1, nb)

                drain(ci, b)

                @pl.when(ci >= 2)
                def _():
                    pltpu.make_async_copy(ov[b], out_hbm.at[pl.ds(base, R)],
                                          osem[b]).wait()

                gb, zb, ob = gat[b], zd[b], ov[b]

                def row(r, _):
                    e = (ci * R + r) * 8
                    vjs = [plsc.load_gather(
                        vv, [jnp.full((16,), e + j, jnp.int32)])
                        for j in range(8)]
                    if diag:
                        dv = plsc.load_gather(
                            dvv, [jnp.full((16,), ci * R + r, jnp.int32)])
                    for wt in range(W // 16):
                        sl = pl.ds(wt * 16, 16)
                        if diag:
                            acc = dv * zb[r, sl]
                        else:
                            acc = vjs[0] * gb[r * 8, sl]
                        for j in range(0 if diag else 1, 8):
                            acc = acc + vjs[j] * gb[r * 8 + j, sl]
                        ob[r, sl] = acc
                    return 0

                lax.fori_loop(0, R, row, 0)
                pltpu.async_copy(ov[b], out_hbm.at[pl.ds(base, R)], osem[b])
            return 0

        lax.fori_loop(0, nchunk // 2, pair, 0)
        for b in range(2):
            last = pl.multiple_of(wbase + (nchunk - 2 + b) * R, 8)
            pltpu.make_async_copy(ov[b], out_hbm.at[pl.ds(last, R)],
                                  osem[b]).wait()

    return k(z, cols, vals, dvals)


def kernel(x_enc0, x_enc1, x_enc2, x_enc3, lap1_idx, lap1_val, lap2_idx,
           lap2_val, lap3_idx, lap3_val, W1a, b1a, g1a, be1a, W1b, b1b, g1b,
           be1b, W2a, b2a, g2a, be2a, W2b, b2b, g2b, be2b, W3a, b3a, g3a,
           be3a, W3b, b3b, g3b, be3b, Wfin, bfin):
    x = x_enc0.transpose(1, 0, 2).reshape(768, B * 256)
    bn_prev = None
    specs = [
        (3072, x_enc1, lap1_idx, lap1_val, W1a, g1a, be1a, W1b, g1b, be1b),
        (12288, x_enc2, lap2_idx, lap2_val, W2a, g2a, be2a, W2b, g2b, be2b),
        (49152, x_enc3, lap3_idx, lap3_val, W3a, g3a, be3a, W3b, g3b, be3b),
    ]
    for V, xe, lidx, lval, Wa, ga, bea, Wb, gb, beb in specs:
        Cp = x.shape[1] // B
        Ce = xe.shape[2]
        Coa, Cob = Wa.shape[2], Wb.shape[2]
        cols = lidx[1, :8 * V]
        vals = lval[:8 * V]
        dvals = lval[8 * V:]

        split = 2 * B * Coa > 512
        if split:
            u02, u1, u2 = _conv_a_matmul(x, xe, Wa[:, :Cp, :], Wa[:, Cp:, :],
                                         V, Cp, Ce, Coa, bn=bn_prev,
                                         split=True)
            s = _spmm_sc(u1, cols, vals, dvals, V, B * Coa)
            s2 = _spmm_sc(u2, cols, vals, dvals, V, B * Coa)
            t = _spmm_sc(s2, cols, vals, dvals, V, B * Coa)
            y, stats = _combine_stats(u02, s, t, V, Coa)
        else:
            u02, u12 = _conv_a_matmul(x, xe, Wa[:, :Cp, :], Wa[:, Cp:, :],
                                      V, Cp, Ce, Coa, bn=bn_prev)
            s = _spmm_sc(u12, cols, vals, dvals, V, 2 * B * Coa)
            t = _spmm_sc(s[:, B * Coa:], cols, vals, dvals, V, B * Coa)
            y, stats = _combine_stats(u02, s, t, V, Coa)

        u02, u12 = _conv_b_matmul(y, stats, ga, bea, Wb, V, Coa, Cob)
        s = _spmm_sc(u12, cols, vals, dvals, V, 2 * B * Cob)
        t = _spmm_sc(s[:, B * Cob:], cols, vals, dvals, V, B * Cob)
        y, stats = _combine_stats(u02, s, t, V, Cob)
        x = y
        bn_prev = (stats, gb, beb)

    V3 = 49152
    wfull = jnp.zeros((B, B * 16), jnp.float32)
    for b in range(B):
        wfull = wfull.at[b, b * 16:(b + 1) * 16].set(Wfin[0])
    stats3, g3, be3 = bn_prev
    out = _final_proj(x, stats3, g3, be3, wfull, bfin.reshape(1, 1), V3)
    return out.reshape(B, 1, V3)


# speedup vs baseline: 1.1219x; 1.0289x over previous
import functools

import jax
import jax.numpy as jnp
from jax import lax
from jax.experimental import pallas as pl
from jax.experimental.pallas import tpu as pltpu
from jax.experimental.pallas import tpu_sc as plsc

B = 2
TV = 2048
NWORK = 32
_SC_BUF = 131072


def _bn_apply(y, st_ref, g_ref, be_ref, n, rows, C):
    mean = st_ref[0] * (1.0 / n)
    var = st_ref[1] * (1.0 / n) - mean * mean
    scale = g_ref[0] * lax.rsqrt(var + 1e-5)
    shift = be_ref[0] - mean * scale
    y3 = y.reshape(rows, B, C)
    o = jnp.maximum(y3 * scale[None, None, :] + shift[None, None, :], 0.0)
    return o.reshape(rows, B * C)


def _conv_a_matmul(xprev, xenc, WT, WB, V, Cp, Ce, Co, bn=None, split=False):
    TV4 = TV // 4
    BCo = B * Co
    n_prev = float((V // 4) * B)

    def body(xp_ref, xe_ref, wt_ref, wb_ref, *refs):
        if bn is not None:
            st_ref, g_ref, be_ref = refs[:3]
            refs = refs[3:]
            xp = _bn_apply(xp_ref[...], st_ref, g_ref, be_ref, n_prev, TV4, Cp)
        else:
            xp = xp_ref[...]
        out_refs = refs
        parts = []
        for k in range(3):
            for b in range(B):
                xpb = xp[:, b * Cp:(b + 1) * Cp]
                xeb = xe_ref[b]
                p = jnp.dot(xpb, wt_ref[k], preferred_element_type=jnp.float32)
                p4 = jnp.broadcast_to(p[:, None, :],
                                      (TV4, 4, Co)).reshape(TV, Co)
                parts.append(p4 + jnp.dot(xeb, wb_ref[k],
                                          preferred_element_type=jnp.float32))
        if split:
            for k in range(3):
                out_refs[k][...] = jnp.concatenate(parts[k * B:(k + 1) * B],
                                                   axis=1)
        else:
            out_refs[0][...] = jnp.concatenate(parts[:B], axis=1)
            out_refs[1][...] = jnp.concatenate(parts[B:], axis=1)

    widths = [BCo, BCo, BCo] if split else [BCo, 2 * BCo]
    in_specs = [
        pl.BlockSpec((TV4, B * Cp), lambda i: (i, 0)),
        pl.BlockSpec((B, TV, Ce), lambda i: (0, i, 0)),
        pl.BlockSpec((3, Cp, Co), lambda i: (0, 0, 0)),
        pl.BlockSpec((3, Ce, Co), lambda i: (0, 0, 0)),
    ]
    args = [xprev, xenc,
            jnp.stack([WT[0] - WT[2], WT[1], WT[2]]),
            jnp.stack([WB[0] - WB[2], WB[1], WB[2]])]
    if bn is not None:
        stats, g, be = bn
        in_specs += [
            pl.BlockSpec((2, Cp), lambda i: (0, 0)),
            pl.BlockSpec((1, Cp), lambda i: (0, 0)),
            pl.BlockSpec((1, Cp), lambda i: (0, 0)),
        ]
        args += [stats, g.reshape(1, Cp), be.reshape(1, Cp)]
    return pl.pallas_call(
        body,
        grid=(V // TV,),
        in_specs=in_specs,
        out_specs=[pl.BlockSpec((TV, w), lambda i: (i, 0)) for w in widths],
        out_shape=[jax.ShapeDtypeStruct((V, w), jnp.float32) for w in widths],
    )(*args)


def _conv_b_matmul(y, stats, g, be, W, V, Cin, Co):
    BCo = B * Co
    n = float(V * B)

    def body(y_ref, st_ref, g_ref, be_ref, w_ref, u0_ref, u12_ref):
        zt = _bn_apply(y_ref[...], st_ref, g_ref, be_ref, n, TV, Cin)
        parts = []
        for k in range(3):
            for b in range(B):
                zb = zt[:, b * Cin:(b + 1) * Cin]
                parts.append(jnp.dot(zb, w_ref[k],
                                     preferred_element_type=jnp.float32))
        u0_ref[...] = jnp.concatenate(parts[:B], axis=1)
        u12_ref[...] = jnp.concatenate(parts[B:], axis=1)

    return pl.pallas_call(
        body,
        grid=(V // TV,),
        in_specs=[
            pl.BlockSpec((TV, B * Cin), lambda i: (i, 0)),
            pl.BlockSpec((2, Cin), lambda i: (0, 0)),
            pl.BlockSpec((1, Cin), lambda i: (0, 0)),
            pl.BlockSpec((1, Cin), lambda i: (0, 0)),
            pl.BlockSpec((3, Cin, Co), lambda i: (0, 0, 0)),
        ],
        out_specs=[
            pl.BlockSpec((TV, BCo), lambda i: (i, 0)),
            pl.BlockSpec((TV, 2 * BCo), lambda i: (i, 0)),
        ],
        out_shape=[
            jax.ShapeDtypeStruct((V, BCo), jnp.float32),
            jax.ShapeDtypeStruct((V, 2 * BCo), jnp.float32),
        ],
    )(y, stats, g.reshape(1, Cin), be.reshape(1, Cin),
      jnp.stack([W[0] - W[2], W[1], W[2]]))


def _combine_stats(u02, s, t, V, Co, dvals=None):
    BCo = B * Co
    Ws = s.shape[1]

    def body(*refs):
        if dvals is not None:
            u0_ref, s_ref, t_ref, d_ref, y_ref, st_ref = refs
            lu2 = s_ref[:, BCo:2 * BCo]
            t_full = t_ref[...] + d_ref[...] * lu2
        else:
            u0_ref, s_ref, t_ref, y_ref, st_ref = refs
            t_full = t_ref[...]
        y = u0_ref[...] + s_ref[:, :BCo] + 2.0 * t_full
        y_ref[...] = y
        y3 = y.reshape(TV, B, Co)
        ssum = jnp.sum(y3, axis=(0, 1))
        ssq = jnp.sum(y3 * y3, axis=(0, 1))
        part = jnp.stack([ssum, ssq], axis=0)

        @pl.when(pl.program_id(0) == 0)
        def _():
            st_ref[...] = jnp.zeros_like(st_ref)

        st_ref[...] += part

    in_specs = [
        pl.BlockSpec((TV, BCo), lambda i: (i, 0)),
        pl.BlockSpec((TV, Ws), lambda i: (i, 0)),
        pl.BlockSpec((TV, BCo), lambda i: (i, 0)),
    ]
    args = [u02, s, t]
    if dvals is not None:
        in_specs.append(pl.BlockSpec((TV, 1), lambda i: (i, 0)))
        args.append(dvals.reshape(V, 1))
    return pl.pallas_call(
        body,
        grid=(V // TV,),
        in_specs=in_specs,
        out_specs=[
            pl.BlockSpec((TV, BCo), lambda i: (i, 0)),
            pl.BlockSpec((2, Co), lambda i: (0, 0)),
        ],
        out_shape=[
            jax.ShapeDtypeStruct((V, BCo), jnp.float32),
            jax.ShapeDtypeStruct((2, Co), jnp.float32),
        ],
    )(*args)


def _final_proj(y, stats, g, be, wfull, bfin, V):
    n = float(V * B)

    def body(y_ref, st_ref, g_ref, be_ref, w_ref, b_ref, o_ref):
        x = _bn_apply(y_ref[...], st_ref, g_ref, be_ref, n, TV, 16)
        r = lax.dot_general(w_ref[...], x, (((1,), (1,)), ((), ())),
                            preferred_element_type=jnp.float32)
        o_ref[...] = r + b_ref[0, 0]

    return pl.pallas_call(
        body,
        grid=(V // TV,),
        in_specs=[
            pl.BlockSpec((TV, B * 16), lambda i: (i, 0)),
            pl.BlockSpec((2, 16), lambda i: (0, 0)),
            pl.BlockSpec((1, 16), lambda i: (0, 0)),
            pl.BlockSpec((1, 16), lambda i: (0, 0)),
            pl.BlockSpec((B, B * 16), lambda i: (0, 0)),
            pl.BlockSpec((1, 1), lambda i: (0, 0)),
        ],
        out_specs=pl.BlockSpec((B, TV), lambda i: (0, i)),
        out_shape=jax.ShapeDtypeStruct((B, V), jnp.float32),
    )(y, stats, g.reshape(1, 16), be.reshape(1, 16), wfull, bfin)


def _spmm_plan(V, W):
    rpw = V // NWORK
    for R in (128, 64, 32, 16, 8):
        if R * 8 * W * 4 <= _SC_BUF and rpw % R == 0:
            break
    G = R * 8
    colw = min(G, 128)
    return rpw, R, G, colw, G // colw


def _spmm_sc(z, cols, vals, dvals, V, W, col_off=0, diag=True):
    rpw, R, G, colw, nG = _spmm_plan(V, W)
    nchunk = rpw // R
    assert nchunk % 2 == 0 and G % colw == 0
    wide = col_off > 0 or z.shape[1] != W
    csl = (pl.ds(col_off, W),) if wide else ()
    mesh = plsc.VectorSubcoreMesh(core_axis_name="c", subcore_axis_name="s")

    @functools.partial(
        pl.kernel, mesh=mesh,
        compiler_params=pltpu.CompilerParams(
            needs_layout_passes=False,
            use_tc_tiling_on_sc=(W % 128 == 0)),
        out_type=jax.ShapeDtypeStruct((V, W), jnp.float32),
        scratch_types=(
            [pltpu.VMEM((rpw * 8,), jnp.int32),
             pltpu.VMEM((rpw * 8,), jnp.float32),
             pltpu.VMEM((rpw,), jnp.float32)]
            + [pltpu.VMEM((G, W), jnp.float32) for _ in range(2)]
            + [pltpu.VMEM((R, W), jnp.float32) for _ in range(4)]
            + [pltpu.SemaphoreType.DMA for _ in range(4)]
        ),
    )
    def k(z_hbm, cols_hbm, vals_hbm, dv_hbm, out_hbm,
          colv, vv, dvv, gat0, gat1, zd0, zd1, ov0, ov1,
          sem0, sem1, osem0, osem1):
        wid = lax.axis_index("s") * 2 + lax.axis_index("c")
        wbase = pl.multiple_of(wid * rpw, 8)
        gat = (gat0, gat1)
        zd = (zd0, zd1)
        ov = (ov0, ov1)
        sem = (sem0, sem1)
        osem = (osem0, osem1)

        pltpu.sync_copy(cols_hbm.at[pl.ds(pl.multiple_of(wbase * 8, 64),
                                          rpw * 8)], colv)
        pltpu.sync_copy(vals_hbm.at[pl.ds(pl.multiple_of(wbase * 8, 64),
                                          rpw * 8)], vv)
        if diag:
            pltpu.sync_copy(dv_hbm.at[pl.ds(wbase, rpw)], dvv)

        def fire(ci, b):
            base = pl.multiple_of(wbase + ci * R, 8)
            for g in range(nG):
                pltpu.async_copy(
                    z_hbm.at[(colv.at[pl.ds(ci * G + g * colw, colw)],) + csl],
                    gat[b].at[pl.ds(g * colw, colw)], sem[b])
            if diag:
                pltpu.async_copy(z_hbm.at[(pl.ds(base, R),) + csl],
                                 zd[b], sem[b])

        def drain(ci, b):
            base = pl.multiple_of(wbase + ci * R, 8)
            for g in range(nG):
                pltpu.make_async_copy(
                    z_hbm.at[(colv.at[pl.ds(ci * G + g * colw, colw)],) + csl],
                    gat[b].at[pl.ds(g * colw, colw)], sem[b]).wait()
            if diag:
                pltpu.make_async_copy(z_hbm.at[(pl.ds(base, R),) + csl],
                                      zd[b], sem[b]).wait()

        fire(0, 0)

        def pair(ii, _):
            for b in range(2):
                ci = ii * 2 + b
                nb = 1 - b
                base = pl.multiple_of(wbase + ci * R, 8)

                @pl.when(ci + 1 < nchunk)
                def _():
                    fire(ci + 1, nb)

                drain(ci, b)

                @pl.when(ci >= 2)
                def _():
                    pltpu.make_async_copy(ov[b], out_hbm.at[pl.ds(base, R)],
                                          osem[b]).wait()

                gb, zb, ob = gat[b], zd[b], ov[b]

                def row(r, _):
                    e = (ci * R + r) * 8
                    vjs = [plsc.load_gather(
                        vv, [jnp.full((16,), e + j, jnp.int32)])
                        for j in range(8)]
                    if diag:
                        dv = plsc.load_gather(
                            dvv, [jnp.full((16,), ci * R + r, jnp.int32)])
                    for wt in range(W // 16):
                        sl = pl.ds(wt * 16, 16)
                        if diag:
                            acc = dv * zb[r, sl]
                        else:
                            acc = vjs[0] * gb[r * 8, sl]
                        for j in range(0 if diag else 1, 8):
                            acc = acc + vjs[j] * gb[r * 8 + j, sl]
                        ob[r, sl] = acc
                    return 0

                lax.fori_loop(0, R, row, 0)
                pltpu.async_copy(ov[b], out_hbm.at[pl.ds(base, R)], osem[b])
            return 0

        lax.fori_loop(0, nchunk // 2, pair, 0)
        for b in range(2):
            last = pl.multiple_of(wbase + (nchunk - 2 + b) * R, 8)
            pltpu.make_async_copy(ov[b], out_hbm.at[pl.ds(last, R)],
                                  osem[b]).wait()

    return k(z, cols, vals, dvals)


def kernel(x_enc0, x_enc1, x_enc2, x_enc3, lap1_idx, lap1_val, lap2_idx,
           lap2_val, lap3_idx, lap3_val, W1a, b1a, g1a, be1a, W1b, b1b, g1b,
           be1b, W2a, b2a, g2a, be2a, W2b, b2b, g2b, be2b, W3a, b3a, g3a,
           be3a, W3b, b3b, g3b, be3b, Wfin, bfin):
    x = x_enc0.transpose(1, 0, 2).reshape(768, B * 256)
    bn_prev = None
    specs = [
        (3072, x_enc1, lap1_idx, lap1_val, W1a, g1a, be1a, W1b, g1b, be1b),
        (12288, x_enc2, lap2_idx, lap2_val, W2a, g2a, be2a, W2b, g2b, be2b),
        (49152, x_enc3, lap3_idx, lap3_val, W3a, g3a, be3a, W3b, g3b, be3b),
    ]
    for V, xe, lidx, lval, Wa, ga, bea, Wb, gb, beb in specs:
        Cp = x.shape[1] // B
        Ce = xe.shape[2]
        Coa, Cob = Wa.shape[2], Wb.shape[2]
        cols = lidx[1, :8 * V]
        vals = lval[:8 * V]
        dvals = lval[8 * V:]

        split = 2 * B * Coa > 512
        if split:
            u02, u1, u2 = _conv_a_matmul(x, xe, Wa[:, :Cp, :], Wa[:, Cp:, :],
                                         V, Cp, Ce, Coa, bn=bn_prev,
                                         split=True)
            s = _spmm_sc(u1, cols, vals, dvals, V, B * Coa)
            s2 = _spmm_sc(u2, cols, vals, dvals, V, B * Coa)
            t = _spmm_sc(s2, cols, vals, dvals, V, B * Coa)
            y, stats = _combine_stats(u02, s, t, V, Coa)
        else:
            u02, u12 = _conv_a_matmul(x, xe, Wa[:, :Cp, :], Wa[:, Cp:, :],
                                      V, Cp, Ce, Coa, bn=bn_prev)
            s = _spmm_sc(u12, cols, vals, dvals, V, 2 * B * Coa)
            t = _spmm_sc(s[:, B * Coa:], cols, vals, dvals, V, B * Coa)
            y, stats = _combine_stats(u02, s, t, V, Coa)

        u02, u12 = _conv_b_matmul(y, stats, ga, bea, Wb, V, Coa, Cob)
        s = _spmm_sc(u12, cols, vals, dvals, V, 2 * B * Cob)
        t = _spmm_sc(s[:, B * Cob:], cols, vals, dvals, V, B * Cob)
        y, stats = _combine_stats(u02, s, t, V, Cob)
        x = y
        bn_prev = (stats, gb, beb)

    V3 = 49152
    wfull = jnp.zeros((B, B * 16), jnp.float32)
    for b in range(B):
        wfull = wfull.at[b, b * 16:(b + 1) * 16].set(Wfin[0])
    stats3, g3, be3 = bn_prev
    out = _final_proj(x, stats3, g3, be3, wfull, bfin.reshape(1, 1), V3)
    return out.reshape(B, 1, V3)
